# Initial kernel scaffold; baseline (speedup 1.0000x reference)
import jax
import jax.numpy as jnp
from jax.experimental import pallas as pl


def kernel(x, edge_index, W1l, b1l, W1r, W2l, b2l, W2r):
    raise NotImplementedError("write your pallas kernel here")


# speedup vs baseline: 7.5438x; 7.5438x over previous
import functools

import jax
import jax.numpy as jnp
from jax import lax
from jax.experimental import pallas as pl
from jax.experimental.pallas import tpu as pltpu
from jax.experimental.pallas import tpu_sc as plsc

_NC, _NS = 2, 16
_NW = _NC * _NS
_CHUNK = 128


def _seg_body(n, e, d, table, src, dst, z_d, sum_out,
              idxs_v, idxd_v, rows_v, acc_sh, sem,
              z_c=None, ones_h=None, cnt_out=None, ones_v=None, cnt_sh=None):
    c = lax.axis_index("c")
    s = lax.axis_index("s")
    wid = s * _NC + c
    rpt = (n // (_NS * 8)) * 8
    tail = n - _NS * rpt
    base_row = s * rpt

    pltpu.sync_copy(z_d, acc_sh.at[pl.ds(base_row, rpt)])
    if cnt_sh is not None:
        pltpu.sync_copy(z_c, cnt_sh.at[pl.ds(base_row, rpt)])
        pltpu.sync_copy(ones_h, ones_v)
    if tail:
        @pl.when(s == 0)
        def _():
            pltpu.sync_copy(z_d.at[pl.ds(0, tail)],
                            acc_sh.at[pl.ds(_NS * rpt, tail)])
            if cnt_sh is not None:
                pltpu.sync_copy(z_c.at[pl.ds(0, tail)],
                                cnt_sh.at[pl.ds(_NS * rpt, tail)])
    plsc.subcore_barrier()

    n_chunks = e // _CHUNK
    iters = -(-n_chunks // _NW)

    def body(j, carry):
        cid = wid + j * _NW

        @pl.when(cid < n_chunks)
        def _():
            off = cid * _CHUNK
            pltpu.sync_copy(src.at[pl.ds(off, _CHUNK)], idxs_v)
            pltpu.sync_copy(dst.at[pl.ds(off, _CHUNK)], idxd_v)
            pltpu.async_copy(table.at[idxs_v], rows_v, sem).wait()
            pltpu.sync_copy(rows_v, acc_sh.at[idxd_v], add=True)
            if cnt_sh is not None:
                pltpu.sync_copy(ones_v, cnt_sh.at[idxd_v], add=True)

        return carry

    lax.fori_loop(0, iters, body, 0)
    plsc.subcore_barrier()

    pltpu.sync_copy(acc_sh.at[pl.ds(base_row, rpt)],
                    sum_out.at[pl.ds(c * n + base_row, rpt)])
    if cnt_sh is not None:
        pltpu.sync_copy(cnt_sh.at[pl.ds(base_row, rpt)],
                        cnt_out.at[pl.ds(c * n + base_row, rpt)])
    if tail:
        @pl.when(s == 0)
        def _():
            pltpu.sync_copy(acc_sh.at[pl.ds(_NS * rpt, tail)],
                            sum_out.at[pl.ds(c * n + _NS * rpt, tail)])
            if cnt_sh is not None:
                pltpu.sync_copy(cnt_sh.at[pl.ds(_NS * rpt, tail)],
                                cnt_out.at[pl.ds(c * n + _NS * rpt, tail)])


@functools.lru_cache(maxsize=None)
def _make_segsum_count(n, e, d):
    mesh = plsc.VectorSubcoreMesh(core_axis_name="c", subcore_axis_name="s")

    @functools.partial(
        pl.kernel,
        out_type=(jax.ShapeDtypeStruct((_NC * n, d), jnp.float32),
                  jax.ShapeDtypeStruct((_NC * n, 16), jnp.float32)),
        mesh=mesh,
        scratch_types=[
            pltpu.VMEM((_CHUNK,), jnp.int32),
            pltpu.VMEM((_CHUNK,), jnp.int32),
            pltpu.VMEM((_CHUNK, d), jnp.float32),
            pltpu.VMEM((_CHUNK, 16), jnp.float32),
            pltpu.VMEM_SHARED((n, d), jnp.float32),
            pltpu.VMEM_SHARED((n, 16), jnp.float32),
            pltpu.SemaphoreType.DMA,
        ],
        compiler_params=pltpu.CompilerParams(use_tc_tiling_on_sc=False),
    )
    def seg(table, src, dst, z_d, z_c, ones_h, sum_out, cnt_out,
            idxs_v, idxd_v, rows_v, ones_v, acc_sh, cnt_sh, sem):
        _seg_body(n, e, d, table, src, dst, z_d, sum_out,
                  idxs_v, idxd_v, rows_v, acc_sh, sem,
                  z_c=z_c, ones_h=ones_h, cnt_out=cnt_out,
                  ones_v=ones_v, cnt_sh=cnt_sh)

    return seg


@functools.lru_cache(maxsize=None)
def _make_segsum(n, e, d):
    mesh = plsc.VectorSubcoreMesh(core_axis_name="c", subcore_axis_name="s")

    @functools.partial(
        pl.kernel,
        out_type=jax.ShapeDtypeStruct((_NC * n, d), jnp.float32),
        mesh=mesh,
        scratch_types=[
            pltpu.VMEM((_CHUNK,), jnp.int32),
            pltpu.VMEM((_CHUNK,), jnp.int32),
            pltpu.VMEM((_CHUNK, d), jnp.float32),
            pltpu.VMEM_SHARED((n, d), jnp.float32),
            pltpu.SemaphoreType.DMA,
        ],
        compiler_params=pltpu.CompilerParams(use_tc_tiling_on_sc=False),
    )
    def seg(table, src, dst, z_d, sum_out, idxs_v, idxd_v, rows_v, acc_sh, sem):
        _seg_body(n, e, d, table, src, dst, z_d, sum_out,
                  idxs_v, idxd_v, rows_v, acc_sh, sem)

    return seg


def _mm_body(x_ref, w_ref, o_ref):
    o_ref[...] = jnp.dot(x_ref[...], w_ref[...],
                         preferred_element_type=jnp.float32)


def _matmul(x, w):
    return pl.pallas_call(
        _mm_body,
        out_shape=jax.ShapeDtypeStruct((x.shape[0], w.shape[1]), jnp.float32),
    )(x, w)


def _layer1(sums, cnts, xr, b):
    n = xr.shape[0]

    def body(s_ref, c_ref, xr_ref, b_ref, o_ref):
        sarr = s_ref[...]
        carr = c_ref[...]
        sm = sarr[:n] + sarr[n:]
        cnt = carr[:n, 0:1] + carr[n:, 0:1]
        o_ref[...] = jnp.maximum(sm / jnp.maximum(cnt, 1.0) + b_ref[...]
                                 + xr_ref[...], 0.0)

    return pl.pallas_call(
        body,
        out_shape=jax.ShapeDtypeStruct(xr.shape, jnp.float32),
    )(sums, cnts, xr, b)


def _layer2(sums, cnts, h, wl, wr, b):
    n = h.shape[0]

    def body(s_ref, c_ref, h_ref, wl_ref, wr_ref, b_ref, o_ref):
        sarr = s_ref[...]
        carr = c_ref[...]
        sm = sarr[:n] + sarr[n:]
        cnt = carr[:n, 0:1] + carr[n:, 0:1]
        a2 = sm / jnp.maximum(cnt, 1.0)
        o = (jnp.dot(a2, wl_ref[...], preferred_element_type=jnp.float32)
             + jnp.dot(h_ref[...], wr_ref[...],
                       preferred_element_type=jnp.float32)
             + b_ref[...])
        m = jnp.max(o, axis=1, keepdims=True)
        lse = jnp.log(jnp.sum(jnp.exp(o - m), axis=1, keepdims=True)) + m
        o_ref[...] = o - lse

    return pl.pallas_call(
        body,
        out_shape=jax.ShapeDtypeStruct((n, wl.shape[1]), jnp.float32),
    )(sums, cnts, h, wl, wr, b)


def kernel(x, edge_index, W1l, b1l, W1r, W2l, b2l, W2r):
    n, _ = x.shape
    d_hid = W1l.shape[1]
    e = edge_index.shape[1]
    src = edge_index[0]
    dst = edge_index[1]

    xcat = _matmul(x, jnp.concatenate([W1l, W1r], axis=1))
    p = xcat[:, :d_hid]
    xr = xcat[:, d_hid:]

    rpt = (n // (_NS * 8)) * 8
    z_d = jnp.zeros((rpt, d_hid), jnp.float32)
    z_c = jnp.zeros((rpt, 16), jnp.float32)
    ones_h = jnp.ones((_CHUNK, 16), jnp.float32)

    sums1, cnts = _make_segsum_count(n, e, d_hid)(p, src, dst, z_d, z_c, ones_h)
    h = _layer1(sums1, cnts, xr, b1l.reshape(1, -1))
    sums2 = _make_segsum(n, e, d_hid)(h, src, dst, z_d)
    return _layer2(sums2, cnts, h, W2l, W2r, b2l.reshape(1, -1))
